# Initial kernel scaffold; baseline (speedup 1.0000x reference)
import jax
import jax.numpy as jnp
from jax.experimental import pallas as pl


def kernel(pos, edge_index, epsilon, sigma):
    raise NotImplementedError("write your pallas kernel here")


# speedup vs baseline: 31.7296x; 31.7296x over previous
import jax
import jax.numpy as jnp
from jax import lax
from jax.experimental import pallas as pl
from jax.experimental.pallas import tpu as pltpu
from jax.experimental.pallas import tpu_sc as plsc

N_NODES = 100000
N_EDGES = 6400000

NW = 32
CH = 2048
NCH = N_EDGES // CH
NP = 100352
SLC = NP // 16
R2 = NP // NW

_f32 = jnp.float32
_i32 = jnp.int32


def _rsqrt(t):
    bits = lax.bitcast_convert_type(t, _i32)
    y = lax.bitcast_convert_type(jnp.int32(0x5F3759DF) - (bits >> 1), _f32)
    for _ in range(3):
        y = y * (1.5 - 0.5 * t * y * y)
    return y


def _edge_pass(es1, ed1, px, py, pz, consts, zeros, fpart, epart,
               si1, di1, xs1, ys1, zs1, xd1, yd1, zd1,
               gsx, gsy, gsz, gdx, gdy, gdz, cbuf, fax, fay, faz):
    c = lax.axis_index("c")
    s = lax.axis_index("s")
    wid = s * 2 + c

    sl = pl.ds(s * SLC, SLC)
    pltpu.sync_copy(zeros.at[sl], fax.at[sl])
    pltpu.sync_copy(zeros.at[sl], fay.at[sl])
    pltpu.sync_copy(zeros.at[sl], faz.at[sl])
    pltpu.sync_copy(consts, cbuf)
    plsc.subcore_barrier()

    eps4 = cbuf[pl.ds(0, 16)]
    sig = cbuf[pl.ds(16, 16)]

    nk = (NCH - wid + NW - 1) // NW

    def chunk_body(k, eacc):
        j = wid + k * NW
        pltpu.sync_copy(es1.at[pl.ds(j * CH, CH)], si1)
        pltpu.sync_copy(ed1.at[pl.ds(j * CH, CH)], di1)
        pltpu.sync_copy(px.at[si1], xs1)
        pltpu.sync_copy(py.at[si1], ys1)
        pltpu.sync_copy(pz.at[si1], zs1)
        pltpu.sync_copy(px.at[di1], xd1)
        pltpu.sync_copy(py.at[di1], yd1)
        pltpu.sync_copy(pz.at[di1], zd1)

        def blk(b, acc):
            cs = pl.ds(b * 16, 16)
            dx = xs1[cs] - xd1[cs]
            dy = ys1[cs] - yd1[cs]
            dz = zs1[cs] - zd1[cs]
            t = dx * dx + dy * dy + dz * dz + 1e-12
            rin = _rsqrt(t)
            rr = t * rin
            qi = 1.0 / (rr + 1.0)
            inv = sig * qi
            i2 = inv * inv
            i6 = i2 * i2 * i2
            i12 = i6 * i6
            acc = acc + eps4 * (i12 - i6)
            dvdr = eps4 * (6.0 * i6 - 12.0 * i12) * qi
            cf = -0.5 * dvdr * rin
            hx = cf * dx
            hy = cf * dy
            hz = cf * dz
            gsx[cs] = hx
            gsy[cs] = hy
            gsz[cs] = hz
            gdx[cs] = -hx
            gdy[cs] = -hy
            gdz[cs] = -hz
            return acc

        eacc = lax.fori_loop(0, CH // 16, blk, eacc)

        pltpu.sync_copy(gsx, fax.at[si1], add=True)
        pltpu.sync_copy(gsy, fay.at[si1], add=True)
        pltpu.sync_copy(gsz, faz.at[si1], add=True)
        pltpu.sync_copy(gdx, fax.at[di1], add=True)
        pltpu.sync_copy(gdy, fay.at[di1], add=True)
        pltpu.sync_copy(gdz, faz.at[di1], add=True)
        return eacc

    eacc = lax.fori_loop(0, nk, chunk_body, jnp.zeros((16,), _f32))

    cbuf[pl.ds(0, 16)] = eacc
    pltpu.sync_copy(cbuf.at[pl.ds(0, 16)], epart.at[pl.ds(wid * 16, 16)])

    plsc.subcore_barrier()
    pltpu.sync_copy(fax.at[sl], fpart.at[pl.ds((c * 3 + 0) * NP + s * SLC, SLC)])
    pltpu.sync_copy(fay.at[sl], fpart.at[pl.ds((c * 3 + 1) * NP + s * SLC, SLC)])
    pltpu.sync_copy(faz.at[sl], fpart.at[pl.ds((c * 3 + 2) * NP + s * SLC, SLC)])


def _combine_pass(fpart, epart, ivec, force, eout,
                  idxb, abuf, bbuf, rows, ebuf):
    c = lax.axis_index("c")
    s = lax.axis_index("s")
    wid = s * 2 + c
    base3 = wid * R2 * 3
    pltpu.sync_copy(ivec.at[pl.ds(base3, R2 * 3)], idxb)
    pltpu.sync_copy(fpart.at[idxb], abuf)

    def mk2(g, carry):
        cs = pl.ds(g * 16, 16)
        idxb[cs] = idxb[cs] + (3 * NP)
        return carry

    lax.fori_loop(0, R2 * 3 // 16, mk2, jnp.int32(0))
    pltpu.sync_copy(fpart.at[idxb], bbuf)

    def blk(g, carry):
        cs = pl.ds(g * 16, 16)
        rows[cs] = abuf[cs] + bbuf[cs]
        return carry

    lax.fori_loop(0, R2 * 3 // 16, blk, jnp.int32(0))
    pltpu.sync_copy(rows, force.at[pl.ds(base3, R2 * 3)])

    @pl.when(wid == 0)
    def _():
        pltpu.sync_copy(epart, ebuf)

        def erow(i, acc):
            return acc + ebuf[pl.ds(i * 16, 16)]

        evec = lax.fori_loop(0, NW, erow, jnp.zeros((16,), _f32))
        ebuf[pl.ds(0, 16)] = evec
        pltpu.sync_copy(ebuf.at[pl.ds(0, 16)], eout)


def kernel(pos, edge_index, epsilon, sigma):
    pos = pos.astype(_f32)
    px = jnp.pad(pos[:, 0], (0, NP - N_NODES))
    py = jnp.pad(pos[:, 1], (0, NP - N_NODES))
    pz = jnp.pad(pos[:, 2], (0, NP - N_NODES))
    es1 = edge_index[0]
    ed1 = edge_index[1]
    consts = jnp.concatenate([jnp.full((16,), 4.0 * epsilon, _f32),
                              jnp.full((16,), sigma, _f32)])
    zeros = jnp.zeros((NP,), _f32)
    w = jnp.arange(NP * 3, dtype=_i32)
    ivec = (w % 3) * NP + w // 3

    mesh = plsc.VectorSubcoreMesh(core_axis_name="c", subcore_axis_name="s")

    fpart, epart = pl.kernel(
        _edge_pass,
        out_type=[
            jax.ShapeDtypeStruct((6 * NP,), _f32),
            jax.ShapeDtypeStruct((NW * 16,), _f32),
        ],
        mesh=mesh,
        scratch_types=[
            pltpu.VMEM((CH,), _i32),
            pltpu.VMEM((CH,), _i32),
        ] + [pltpu.VMEM((CH,), _f32)] * 12
        + [
            pltpu.VMEM((32,), _f32),
            pltpu.VMEM_SHARED((NP,), _f32),
            pltpu.VMEM_SHARED((NP,), _f32),
            pltpu.VMEM_SHARED((NP,), _f32),
        ],
    )(es1, ed1, px, py, pz, consts, zeros)

    force, eout = pl.kernel(
        _combine_pass,
        out_type=[
            jax.ShapeDtypeStruct((NP * 3,), _f32),
            jax.ShapeDtypeStruct((16,), _f32),
        ],
        mesh=mesh,
        scratch_types=[
            pltpu.VMEM((R2 * 3,), _i32),
            pltpu.VMEM((R2 * 3,), _f32),
            pltpu.VMEM((R2 * 3,), _f32),
            pltpu.VMEM((R2 * 3,), _f32),
            pltpu.VMEM((NW * 16,), _f32),
        ],
    )(fpart, epart, ivec)

    return jnp.sum(eout), force[:N_NODES * 3].reshape(N_NODES, 3)


# speedup vs baseline: 45.4751x; 1.4332x over previous
import jax
import jax.numpy as jnp
from jax import lax
from jax.experimental import pallas as pl
from jax.experimental.pallas import tpu as pltpu
from jax.experimental.pallas import tpu_sc as plsc

N_NODES = 100000
N_EDGES = 6400000

NW = 32
CH = 2048
NCH = N_EDGES // CH
NP = 100352
SLC = NP // 16
R2 = NP // NW

_f32 = jnp.float32
_i32 = jnp.int32


def _rsqrt(t):
    bits = lax.bitcast_convert_type(t, _i32)
    y = lax.bitcast_convert_type(jnp.int32(0x5F3759DF) - (bits >> 1), _f32)
    for _ in range(3):
        y = y * (1.5 - 0.5 * t * y * y)
    return y


def _edge_pass(es1, ed1, px, py, pz, consts, zeros, fpart, epart,
               cat0, bx0, by0, bz0, gx0, gy0, gz0,
               cat1, bx1, by1, bz1, gx1, gy1, gz1,
               cbuf, fax, fay, faz,
               sg0, sg1, ss0, ss1, sj0, sj1):
    c = lax.axis_index("c")
    s = lax.axis_index("s")
    wid = s * 2 + c

    set0 = (cat0, bx0, by0, bz0, gx0, gy0, gz0, sg0, ss0, sj0)
    set1 = (cat1, bx1, by1, bz1, gx1, gy1, gz1, sg1, ss1, sj1)

    sl = pl.ds(s * SLC, SLC)
    pltpu.sync_copy(zeros.at[sl], fax.at[sl])
    pltpu.sync_copy(zeros.at[sl], fay.at[sl])
    pltpu.sync_copy(zeros.at[sl], faz.at[sl])
    pltpu.sync_copy(consts, cbuf.at[pl.ds(0, 32)])
    cbuf[pl.ds(32, 16)] = jnp.zeros((16,), _f32)
    plsc.subcore_barrier()

    eps4 = cbuf[pl.ds(0, 16)]
    sig = cbuf[pl.ds(16, 16)]

    nk = (NCH - wid + NW - 1) // NW

    def gather_descs(S):
        cat, bx, by, bz = S[0], S[1], S[2], S[3]
        return [pltpu.make_async_copy(px.at[cat], bx, S[7]),
                pltpu.make_async_copy(py.at[cat], by, S[7]),
                pltpu.make_async_copy(pz.at[cat], bz, S[7])]

    def scatter_drain(S):
        cat, gx, gy, gz = S[0], S[4], S[5], S[6]
        pltpu.make_async_copy(gx, fax.at[cat], S[8]).wait()
        pltpu.make_async_copy(gy, fay.at[cat], S[8]).wait()
        pltpu.make_async_copy(gz, faz.at[cat], S[8]).wait()

    def scatter_issue(S):
        cat, gx, gy, gz = S[0], S[4], S[5], S[6]
        pltpu.async_copy(gx, fax.at[cat], S[8], add=True)
        pltpu.async_copy(gy, fay.at[cat], S[8], add=True)
        pltpu.async_copy(gz, faz.at[cat], S[8], add=True)

    def idx_issue(S, j):
        cat = S[0]
        pltpu.async_copy(es1.at[pl.ds(j * CH, CH)], cat.at[pl.ds(0, CH)], S[9])
        pltpu.async_copy(ed1.at[pl.ds(j * CH, CH)], cat.at[pl.ds(CH, CH)], S[9])

    def idx_drain(S):
        cat = S[0]
        pltpu.make_async_copy(es1.at[pl.ds(0, CH)], cat.at[pl.ds(0, CH)], S[9]).wait()
        pltpu.make_async_copy(ed1.at[pl.ds(0, CH)], cat.at[pl.ds(CH, CH)], S[9]).wait()

    def compute(S):
        bx, by, bz, gx, gy, gz = S[1], S[2], S[3], S[4], S[5], S[6]

        def blk(b, acc):
            cs = pl.ds(b * 16, 16)
            cd = pl.ds(CH + b * 16, 16)
            dx = bx[cs] - bx[cd]
            dy = by[cs] - by[cd]
            dz = bz[cs] - bz[cd]
            t = dx * dx + dy * dy + dz * dz + 1e-12
            rin = _rsqrt(t)
            rr = t * rin
            qi = 1.0 / (rr + 1.0)
            inv = sig * qi
            i2 = inv * inv
            i6 = i2 * i2 * i2
            i12 = i6 * i6
            acc = acc + eps4 * (i12 - i6)
            dvdr = eps4 * (6.0 * i6 - 12.0 * i12) * qi
            cf = -0.5 * dvdr * rin
            hx = cf * dx
            hy = cf * dy
            hz = cf * dz
            gx[cs] = hx
            gy[cs] = hy
            gz[cs] = hz
            gx[cd] = -hx
            gy[cd] = -hy
            gz[cd] = -hz
            return acc

        eacc = lax.fori_loop(0, CH // 16, blk, cbuf[pl.ds(32, 16)])
        cbuf[pl.ds(32, 16)] = eacc

    def chunk_ops(k, S, T):
        @pl.when(k >= 1)
        def _():
            scatter_drain(T)

        @pl.when(k + 1 < nk)
        def _():
            idx_issue(T, wid + (k + 1) * NW)

        for d in gather_descs(S):
            d.wait()
        compute(S)
        scatter_issue(S)

        @pl.when(k + 1 < nk)
        def _():
            idx_drain(T)
            for d in gather_descs(T):
                d.start()

    pltpu.sync_copy(es1.at[pl.ds(wid * CH, CH)], cat0.at[pl.ds(0, CH)])
    pltpu.sync_copy(ed1.at[pl.ds(wid * CH, CH)], cat0.at[pl.ds(CH, CH)])
    for d in gather_descs(set0):
        d.start()

    def body(k, carry):
        @pl.when((k & 1) == 0)
        def _():
            chunk_ops(k, set0, set1)

        @pl.when((k & 1) == 1)
        def _():
            chunk_ops(k, set1, set0)

        return carry

    lax.fori_loop(0, nk, body, jnp.int32(0))

    last = (nk - 1) & 1

    @pl.when(last == 0)
    def _():
        scatter_drain(set0)

    @pl.when(last == 1)
    def _():
        scatter_drain(set1)

    pltpu.sync_copy(cbuf.at[pl.ds(32, 16)], epart.at[pl.ds(wid * 16, 16)])

    plsc.subcore_barrier()
    pltpu.sync_copy(fax.at[sl], fpart.at[pl.ds((c * 3 + 0) * NP + s * SLC, SLC)])
    pltpu.sync_copy(fay.at[sl], fpart.at[pl.ds((c * 3 + 1) * NP + s * SLC, SLC)])
    pltpu.sync_copy(faz.at[sl], fpart.at[pl.ds((c * 3 + 2) * NP + s * SLC, SLC)])


def _combine_pass(fpart, epart, ivec, force, eout,
                  idxb, abuf, bbuf, rows, ebuf):
    c = lax.axis_index("c")
    s = lax.axis_index("s")
    wid = s * 2 + c
    base3 = wid * R2 * 3
    pltpu.sync_copy(ivec.at[pl.ds(base3, R2 * 3)], idxb)
    pltpu.sync_copy(fpart.at[idxb], abuf)

    def mk2(g, carry):
        cs = pl.ds(g * 16, 16)
        idxb[cs] = idxb[cs] + (3 * NP)
        return carry

    lax.fori_loop(0, R2 * 3 // 16, mk2, jnp.int32(0))
    pltpu.sync_copy(fpart.at[idxb], bbuf)

    def blk(g, carry):
        cs = pl.ds(g * 16, 16)
        rows[cs] = abuf[cs] + bbuf[cs]
        return carry

    lax.fori_loop(0, R2 * 3 // 16, blk, jnp.int32(0))
    pltpu.sync_copy(rows, force.at[pl.ds(base3, R2 * 3)])

    @pl.when(wid == 0)
    def _():
        pltpu.sync_copy(epart, ebuf)

        def erow(i, acc):
            return acc + ebuf[pl.ds(i * 16, 16)]

        evec = lax.fori_loop(0, NW, erow, jnp.zeros((16,), _f32))
        ebuf[pl.ds(0, 16)] = evec
        pltpu.sync_copy(ebuf.at[pl.ds(0, 16)], eout)


def kernel(pos, edge_index, epsilon, sigma):
    pos = pos.astype(_f32)
    px = jnp.pad(pos[:, 0], (0, NP - N_NODES))
    py = jnp.pad(pos[:, 1], (0, NP - N_NODES))
    pz = jnp.pad(pos[:, 2], (0, NP - N_NODES))
    es1 = edge_index[0]
    ed1 = edge_index[1]
    consts = jnp.concatenate([jnp.full((16,), 4.0 * epsilon, _f32),
                              jnp.full((16,), sigma, _f32)])
    zeros = jnp.zeros((NP,), _f32)
    w = jnp.arange(NP * 3, dtype=_i32)
    ivec = (w % 3) * NP + w // 3

    mesh = plsc.VectorSubcoreMesh(core_axis_name="c", subcore_axis_name="s")

    fpart, epart = pl.kernel(
        _edge_pass,
        out_type=[
            jax.ShapeDtypeStruct((6 * NP,), _f32),
            jax.ShapeDtypeStruct((NW * 16,), _f32),
        ],
        mesh=mesh,
        scratch_types=(
            [pltpu.VMEM((2 * CH,), _i32)]
            + [pltpu.VMEM((2 * CH,), _f32)] * 6
            + [pltpu.VMEM((2 * CH,), _i32)]
            + [pltpu.VMEM((2 * CH,), _f32)] * 6
            + [
                pltpu.VMEM((48,), _f32),
                pltpu.VMEM_SHARED((NP,), _f32),
                pltpu.VMEM_SHARED((NP,), _f32),
                pltpu.VMEM_SHARED((NP,), _f32),
            ]
            + [pltpu.SemaphoreType.DMA] * 6
        ),
    )(es1, ed1, px, py, pz, consts, zeros)

    force, eout = pl.kernel(
        _combine_pass,
        out_type=[
            jax.ShapeDtypeStruct((NP * 3,), _f32),
            jax.ShapeDtypeStruct((16,), _f32),
        ],
        mesh=mesh,
        scratch_types=[
            pltpu.VMEM((R2 * 3,), _i32),
            pltpu.VMEM((R2 * 3,), _f32),
            pltpu.VMEM((R2 * 3,), _f32),
            pltpu.VMEM((R2 * 3,), _f32),
            pltpu.VMEM((NW * 16,), _f32),
        ],
    )(fpart, epart, ivec)

    return jnp.sum(eout), force[:N_NODES * 3].reshape(N_NODES, 3)


# speedup vs baseline: 65.0968x; 1.4315x over previous
import jax
import jax.numpy as jnp
from jax import lax
from jax.experimental import pallas as pl
from jax.experimental.pallas import tpu as pltpu
from jax.experimental.pallas import tpu_sc as plsc

N_NODES = 100000
N_EDGES = 6400000

NW = 32
CH = 2048
NCH = N_EDGES // CH
NP = 100352
SLC = NP // 16
R2 = NP // NW

_f32 = jnp.float32
_i32 = jnp.int32


def _rsqrt(t):
    bits = lax.bitcast_convert_type(t, _i32)
    y = lax.bitcast_convert_type(jnp.int32(0x5F3759DF) - (bits >> 1), _f32)
    for _ in range(3):
        y = y * (1.5 - 0.5 * t * y * y)
    return y


def _edge_pass(es1, ed1, px, py, pz, consts, zeros, fpart, epart,
               cat0, bx0, by0, bz0, gx0, gy0, gz0,
               cat1, bx1, by1, bz1, gx1, gy1, gz1,
               cbuf, fax, fay, faz, spx, spy, spz,
               sg0, sg1, ss0, ss1, sj0, sj1):
    c = lax.axis_index("c")
    s = lax.axis_index("s")
    wid = s * 2 + c

    set0 = (cat0, bx0, by0, bz0, gx0, gy0, gz0, sg0, ss0, sj0)
    set1 = (cat1, bx1, by1, bz1, gx1, gy1, gz1, sg1, ss1, sj1)

    sl = pl.ds(s * SLC, SLC)
    pltpu.sync_copy(zeros.at[sl], fax.at[sl])
    pltpu.sync_copy(zeros.at[sl], fay.at[sl])
    pltpu.sync_copy(zeros.at[sl], faz.at[sl])
    pltpu.sync_copy(px.at[sl], spx.at[sl])
    pltpu.sync_copy(py.at[sl], spy.at[sl])
    pltpu.sync_copy(pz.at[sl], spz.at[sl])
    pltpu.sync_copy(consts, cbuf.at[pl.ds(0, 32)])
    cbuf[pl.ds(32, 16)] = jnp.zeros((16,), _f32)
    plsc.subcore_barrier()

    eps4 = cbuf[pl.ds(0, 16)]
    sig = cbuf[pl.ds(16, 16)]

    nk = (NCH - wid + NW - 1) // NW

    def gather_descs(S):
        cat, bx, by, bz = S[0], S[1], S[2], S[3]
        return [pltpu.make_async_copy(spx.at[cat], bx, S[7]),
                pltpu.make_async_copy(spy.at[cat], by, S[7]),
                pltpu.make_async_copy(spz.at[cat], bz, S[7])]

    def scatter_drain(S):
        cat, gx, gy, gz = S[0], S[4], S[5], S[6]
        pltpu.make_async_copy(gx, fax.at[cat], S[8]).wait()
        pltpu.make_async_copy(gy, fay.at[cat], S[8]).wait()
        pltpu.make_async_copy(gz, faz.at[cat], S[8]).wait()

    def scatter_issue(S):
        cat, gx, gy, gz = S[0], S[4], S[5], S[6]
        pltpu.async_copy(gx, fax.at[cat], S[8], add=True)
        pltpu.async_copy(gy, fay.at[cat], S[8], add=True)
        pltpu.async_copy(gz, faz.at[cat], S[8], add=True)

    def idx_issue(S, j):
        cat = S[0]
        pltpu.async_copy(es1.at[pl.ds(j * CH, CH)], cat.at[pl.ds(0, CH)], S[9])
        pltpu.async_copy(ed1.at[pl.ds(j * CH, CH)], cat.at[pl.ds(CH, CH)], S[9])

    def idx_drain(S):
        cat = S[0]
        pltpu.make_async_copy(es1.at[pl.ds(0, CH)], cat.at[pl.ds(0, CH)], S[9]).wait()
        pltpu.make_async_copy(ed1.at[pl.ds(0, CH)], cat.at[pl.ds(CH, CH)], S[9]).wait()

    def compute(S):
        bx, by, bz, gx, gy, gz = S[1], S[2], S[3], S[4], S[5], S[6]

        def blk(b, acc):
            cs = pl.ds(b * 16, 16)
            cd = pl.ds(CH + b * 16, 16)
            dx = bx[cs] - bx[cd]
            dy = by[cs] - by[cd]
            dz = bz[cs] - bz[cd]
            t = dx * dx + dy * dy + dz * dz + 1e-12
            rin = _rsqrt(t)
            rr = t * rin
            qi = 1.0 / (rr + 1.0)
            inv = sig * qi
            i2 = inv * inv
            i6 = i2 * i2 * i2
            i12 = i6 * i6
            acc = acc + eps4 * (i12 - i6)
            dvdr = eps4 * (6.0 * i6 - 12.0 * i12) * qi
            cf = -0.5 * dvdr * rin
            hx = cf * dx
            hy = cf * dy
            hz = cf * dz
            gx[cs] = hx
            gy[cs] = hy
            gz[cs] = hz
            gx[cd] = -hx
            gy[cd] = -hy
            gz[cd] = -hz
            return acc

        eacc = lax.fori_loop(0, CH // 16, blk, cbuf[pl.ds(32, 16)])
        cbuf[pl.ds(32, 16)] = eacc

    def chunk_ops(k, S, T):
        @pl.when(k >= 1)
        def _():
            scatter_drain(T)

        @pl.when(k + 1 < nk)
        def _():
            idx_issue(T, wid + (k + 1) * NW)

        for d in gather_descs(S):
            d.wait()
        compute(S)
        scatter_issue(S)

        @pl.when(k + 1 < nk)
        def _():
            idx_drain(T)
            for d in gather_descs(T):
                d.start()

    pltpu.sync_copy(es1.at[pl.ds(wid * CH, CH)], cat0.at[pl.ds(0, CH)])
    pltpu.sync_copy(ed1.at[pl.ds(wid * CH, CH)], cat0.at[pl.ds(CH, CH)])
    for d in gather_descs(set0):
        d.start()

    def body(k, carry):
        @pl.when((k & 1) == 0)
        def _():
            chunk_ops(k, set0, set1)

        @pl.when((k & 1) == 1)
        def _():
            chunk_ops(k, set1, set0)

        return carry

    lax.fori_loop(0, nk, body, jnp.int32(0))

    last = (nk - 1) & 1

    @pl.when(last == 0)
    def _():
        scatter_drain(set0)

    @pl.when(last == 1)
    def _():
        scatter_drain(set1)

    pltpu.sync_copy(cbuf.at[pl.ds(32, 16)], epart.at[pl.ds(wid * 16, 16)])

    plsc.subcore_barrier()
    pltpu.sync_copy(fax.at[sl], fpart.at[pl.ds((c * 3 + 0) * NP + s * SLC, SLC)])
    pltpu.sync_copy(fay.at[sl], fpart.at[pl.ds((c * 3 + 1) * NP + s * SLC, SLC)])
    pltpu.sync_copy(faz.at[sl], fpart.at[pl.ds((c * 3 + 2) * NP + s * SLC, SLC)])


def _combine_pass(fpart, epart, ivec, force, eout,
                  idxb, abuf, bbuf, rows, ebuf):
    c = lax.axis_index("c")
    s = lax.axis_index("s")
    wid = s * 2 + c
    base3 = wid * R2 * 3
    pltpu.sync_copy(ivec.at[pl.ds(base3, R2 * 3)], idxb)
    pltpu.sync_copy(fpart.at[idxb], abuf)

    def mk2(g, carry):
        cs = pl.ds(g * 16, 16)
        idxb[cs] = idxb[cs] + (3 * NP)
        return carry

    lax.fori_loop(0, R2 * 3 // 16, mk2, jnp.int32(0))
    pltpu.sync_copy(fpart.at[idxb], bbuf)

    def blk(g, carry):
        cs = pl.ds(g * 16, 16)
        rows[cs] = abuf[cs] + bbuf[cs]
        return carry

    lax.fori_loop(0, R2 * 3 // 16, blk, jnp.int32(0))
    pltpu.sync_copy(rows, force.at[pl.ds(base3, R2 * 3)])

    @pl.when(wid == 0)
    def _():
        pltpu.sync_copy(epart, ebuf)

        def erow(i, acc):
            return acc + ebuf[pl.ds(i * 16, 16)]

        evec = lax.fori_loop(0, NW, erow, jnp.zeros((16,), _f32))
        ebuf[pl.ds(0, 16)] = evec
        pltpu.sync_copy(ebuf.at[pl.ds(0, 16)], eout)


def kernel(pos, edge_index, epsilon, sigma):
    pos = pos.astype(_f32)
    px = jnp.pad(pos[:, 0], (0, NP - N_NODES))
    py = jnp.pad(pos[:, 1], (0, NP - N_NODES))
    pz = jnp.pad(pos[:, 2], (0, NP - N_NODES))
    es1 = edge_index[0]
    ed1 = edge_index[1]
    consts = jnp.concatenate([jnp.full((16,), 4.0 * epsilon, _f32),
                              jnp.full((16,), sigma, _f32)])
    zeros = jnp.zeros((NP,), _f32)
    w = jnp.arange(NP * 3, dtype=_i32)
    ivec = (w % 3) * NP + w // 3

    mesh = plsc.VectorSubcoreMesh(core_axis_name="c", subcore_axis_name="s")

    fpart, epart = pl.kernel(
        _edge_pass,
        out_type=[
            jax.ShapeDtypeStruct((6 * NP,), _f32),
            jax.ShapeDtypeStruct((NW * 16,), _f32),
        ],
        mesh=mesh,
        scratch_types=(
            [pltpu.VMEM((2 * CH,), _i32)]
            + [pltpu.VMEM((2 * CH,), _f32)] * 6
            + [pltpu.VMEM((2 * CH,), _i32)]
            + [pltpu.VMEM((2 * CH,), _f32)] * 6
            + [
                pltpu.VMEM((48,), _f32),
                pltpu.VMEM_SHARED((NP,), _f32),
                pltpu.VMEM_SHARED((NP,), _f32),
                pltpu.VMEM_SHARED((NP,), _f32),
                pltpu.VMEM_SHARED((NP,), _f32),
                pltpu.VMEM_SHARED((NP,), _f32),
                pltpu.VMEM_SHARED((NP,), _f32),
            ]
            + [pltpu.SemaphoreType.DMA] * 6
        ),
    )(es1, ed1, px, py, pz, consts, zeros)

    force, eout = pl.kernel(
        _combine_pass,
        out_type=[
            jax.ShapeDtypeStruct((NP * 3,), _f32),
            jax.ShapeDtypeStruct((16,), _f32),
        ],
        mesh=mesh,
        scratch_types=[
            pltpu.VMEM((R2 * 3,), _i32),
            pltpu.VMEM((R2 * 3,), _f32),
            pltpu.VMEM((R2 * 3,), _f32),
            pltpu.VMEM((R2 * 3,), _f32),
            pltpu.VMEM((NW * 16,), _f32),
        ],
    )(fpart, epart, ivec)

    return jnp.sum(eout), force[:N_NODES * 3].reshape(N_NODES, 3)


# speedup vs baseline: 81.2611x; 1.2483x over previous
import jax
import jax.numpy as jnp
from jax import lax
from jax.experimental import pallas as pl
from jax.experimental.pallas import tpu as pltpu
from jax.experimental.pallas import tpu_sc as plsc

N_NODES = 100000
N_EDGES = 6400000

NW = 32
CH = 2048
NCH = N_EDGES // CH
NP = 100352
SLC = NP // 16
R2 = NP // NW

_f32 = jnp.float32
_i32 = jnp.int32


def _rsqrt(t):
    bits = lax.bitcast_convert_type(t, _i32)
    y = lax.bitcast_convert_type(jnp.int32(0x5F3759DF) - (bits >> 1), _f32)
    for _ in range(3):
        y = y * (1.5 - 0.5 * t * y * y)
    return y


def _edge_pass(es1, ed1, px, py, pz, consts, zeros, fpart, epart,
               cat0, bx0, by0, bz0, gx0, gy0, gz0,
               cat1, bx1, by1, bz1, gx1, gy1, gz1,
               cbuf, fax, fay, faz, spx, spy, spz,
               sg0, sg1, ss0, ss1, sj0, sj1):
    c = lax.axis_index("c")
    s = lax.axis_index("s")
    wid = s * 2 + c

    set0 = (cat0, bx0, by0, bz0, gx0, gy0, gz0, sg0, ss0, sj0)
    set1 = (cat1, bx1, by1, bz1, gx1, gy1, gz1, sg1, ss1, sj1)

    sl = pl.ds(s * SLC, SLC)
    pltpu.sync_copy(zeros.at[sl], fax.at[sl])
    pltpu.sync_copy(zeros.at[sl], fay.at[sl])
    pltpu.sync_copy(zeros.at[sl], faz.at[sl])
    pltpu.sync_copy(px.at[sl], spx.at[sl])
    pltpu.sync_copy(py.at[sl], spy.at[sl])
    pltpu.sync_copy(pz.at[sl], spz.at[sl])
    pltpu.sync_copy(consts, cbuf.at[pl.ds(0, 32)])
    cbuf[pl.ds(32, 16)] = jnp.zeros((16,), _f32)
    plsc.subcore_barrier()

    eps4 = cbuf[pl.ds(0, 16)]
    sig = cbuf[pl.ds(16, 16)]

    nk = (NCH - wid + NW - 1) // NW

    def gather_descs(S):
        cat, bx, by, bz = S[0], S[1], S[2], S[3]
        return [pltpu.make_async_copy(spx.at[cat], bx, S[7]),
                pltpu.make_async_copy(spy.at[cat], by, S[7]),
                pltpu.make_async_copy(spz.at[cat], bz, S[7])]

    def scatter_drain(S):
        cat, gx, gy, gz = S[0], S[4], S[5], S[6]
        pltpu.make_async_copy(gx, fax.at[cat], S[8]).wait()
        pltpu.make_async_copy(gy, fay.at[cat], S[8]).wait()
        pltpu.make_async_copy(gz, faz.at[cat], S[8]).wait()

    def scatter_issue(S):
        cat, gx, gy, gz = S[0], S[4], S[5], S[6]
        pltpu.async_copy(gx, fax.at[cat], S[8], add=True)
        pltpu.async_copy(gy, fay.at[cat], S[8], add=True)
        pltpu.async_copy(gz, faz.at[cat], S[8], add=True)

    def idx_issue(S, j):
        cat = S[0]
        pltpu.async_copy(es1.at[pl.ds(j * CH, CH)], cat.at[pl.ds(0, CH)], S[9])
        pltpu.async_copy(ed1.at[pl.ds(j * CH, CH)], cat.at[pl.ds(CH, CH)], S[9])

    def idx_drain(S):
        cat = S[0]
        pltpu.make_async_copy(es1.at[pl.ds(0, CH)], cat.at[pl.ds(0, CH)], S[9]).wait()
        pltpu.make_async_copy(ed1.at[pl.ds(0, CH)], cat.at[pl.ds(CH, CH)], S[9]).wait()

    def compute(S):
        bx, by, bz, gx, gy, gz = S[1], S[2], S[3], S[4], S[5], S[6]

        def blk(b, acc):
            cs = pl.ds(b * 16, 16)
            cd = pl.ds(CH + b * 16, 16)
            dx = bx[cs] - bx[cd]
            dy = by[cs] - by[cd]
            dz = bz[cs] - bz[cd]
            t = dx * dx + dy * dy + dz * dz + 1e-12
            rin = _rsqrt(t)
            rr = t * rin
            qi = 1.0 / (rr + 1.0)
            inv = sig * qi
            i2 = inv * inv
            i6 = i2 * i2 * i2
            i12 = i6 * i6
            acc = acc + eps4 * (i12 - i6)
            dvdr = eps4 * (6.0 * i6 - 12.0 * i12) * qi
            cf = -0.5 * dvdr * rin
            hx = cf * dx
            hy = cf * dy
            hz = cf * dz
            gx[cs] = hx
            gy[cs] = hy
            gz[cs] = hz
            gx[cd] = -hx
            gy[cd] = -hy
            gz[cd] = -hz
            return acc

        eacc = lax.fori_loop(0, CH // 16, blk, cbuf[pl.ds(32, 16)])
        cbuf[pl.ds(32, 16)] = eacc

    def chunk_ops(k, S, T):
        @pl.when(k >= 1)
        def _():
            scatter_drain(T)

        @pl.when(k + 1 < nk)
        def _():
            idx_issue(T, wid + (k + 1) * NW)

        for d in gather_descs(S):
            d.wait()

        @pl.when(k + 1 < nk)
        def _():
            idx_drain(T)
            for d in gather_descs(T):
                d.start()

        compute(S)
        scatter_issue(S)

    pltpu.sync_copy(es1.at[pl.ds(wid * CH, CH)], cat0.at[pl.ds(0, CH)])
    pltpu.sync_copy(ed1.at[pl.ds(wid * CH, CH)], cat0.at[pl.ds(CH, CH)])
    for d in gather_descs(set0):
        d.start()

    def body(k, carry):
        @pl.when((k & 1) == 0)
        def _():
            chunk_ops(k, set0, set1)

        @pl.when((k & 1) == 1)
        def _():
            chunk_ops(k, set1, set0)

        return carry

    lax.fori_loop(0, nk, body, jnp.int32(0))

    last = (nk - 1) & 1

    @pl.when(last == 0)
    def _():
        scatter_drain(set0)

    @pl.when(last == 1)
    def _():
        scatter_drain(set1)

    pltpu.sync_copy(cbuf.at[pl.ds(32, 16)], epart.at[pl.ds(wid * 16, 16)])

    plsc.subcore_barrier()
    pltpu.sync_copy(fax.at[sl], fpart.at[pl.ds((c * 3 + 0) * NP + s * SLC, SLC)])
    pltpu.sync_copy(fay.at[sl], fpart.at[pl.ds((c * 3 + 1) * NP + s * SLC, SLC)])
    pltpu.sync_copy(faz.at[sl], fpart.at[pl.ds((c * 3 + 2) * NP + s * SLC, SLC)])


def _combine_pass(fpart, epart, ivec, force, eout,
                  idxb, abuf, bbuf, rows, ebuf):
    c = lax.axis_index("c")
    s = lax.axis_index("s")
    wid = s * 2 + c
    base3 = wid * R2 * 3
    pltpu.sync_copy(ivec.at[pl.ds(base3, R2 * 3)], idxb)
    pltpu.sync_copy(fpart.at[idxb], abuf)

    def mk2(g, carry):
        cs = pl.ds(g * 16, 16)
        idxb[cs] = idxb[cs] + (3 * NP)
        return carry

    lax.fori_loop(0, R2 * 3 // 16, mk2, jnp.int32(0))
    pltpu.sync_copy(fpart.at[idxb], bbuf)

    def blk(g, carry):
        cs = pl.ds(g * 16, 16)
        rows[cs] = abuf[cs] + bbuf[cs]
        return carry

    lax.fori_loop(0, R2 * 3 // 16, blk, jnp.int32(0))
    pltpu.sync_copy(rows, force.at[pl.ds(base3, R2 * 3)])

    @pl.when(wid == 0)
    def _():
        pltpu.sync_copy(epart, ebuf)

        def erow(i, acc):
            return acc + ebuf[pl.ds(i * 16, 16)]

        evec = lax.fori_loop(0, NW, erow, jnp.zeros((16,), _f32))
        ebuf[pl.ds(0, 16)] = evec
        pltpu.sync_copy(ebuf.at[pl.ds(0, 16)], eout)


def kernel(pos, edge_index, epsilon, sigma):
    pos = pos.astype(_f32)
    px = jnp.pad(pos[:, 0], (0, NP - N_NODES))
    py = jnp.pad(pos[:, 1], (0, NP - N_NODES))
    pz = jnp.pad(pos[:, 2], (0, NP - N_NODES))
    es1 = edge_index[0]
    ed1 = edge_index[1]
    consts = jnp.concatenate([jnp.full((16,), 4.0 * epsilon, _f32),
                              jnp.full((16,), sigma, _f32)])
    zeros = jnp.zeros((NP,), _f32)
    w = jnp.arange(NP * 3, dtype=_i32)
    ivec = (w % 3) * NP + w // 3

    mesh = plsc.VectorSubcoreMesh(core_axis_name="c", subcore_axis_name="s")

    fpart, epart = pl.kernel(
        _edge_pass,
        out_type=[
            jax.ShapeDtypeStruct((6 * NP,), _f32),
            jax.ShapeDtypeStruct((NW * 16,), _f32),
        ],
        mesh=mesh,
        scratch_types=(
            [pltpu.VMEM((2 * CH,), _i32)]
            + [pltpu.VMEM((2 * CH,), _f32)] * 6
            + [pltpu.VMEM((2 * CH,), _i32)]
            + [pltpu.VMEM((2 * CH,), _f32)] * 6
            + [
                pltpu.VMEM((48,), _f32),
                pltpu.VMEM_SHARED((NP,), _f32),
                pltpu.VMEM_SHARED((NP,), _f32),
                pltpu.VMEM_SHARED((NP,), _f32),
                pltpu.VMEM_SHARED((NP,), _f32),
                pltpu.VMEM_SHARED((NP,), _f32),
                pltpu.VMEM_SHARED((NP,), _f32),
            ]
            + [pltpu.SemaphoreType.DMA] * 6
        ),
    )(es1, ed1, px, py, pz, consts, zeros)

    force, eout = pl.kernel(
        _combine_pass,
        out_type=[
            jax.ShapeDtypeStruct((NP * 3,), _f32),
            jax.ShapeDtypeStruct((16,), _f32),
        ],
        mesh=mesh,
        scratch_types=[
            pltpu.VMEM((R2 * 3,), _i32),
            pltpu.VMEM((R2 * 3,), _f32),
            pltpu.VMEM((R2 * 3,), _f32),
            pltpu.VMEM((R2 * 3,), _f32),
            pltpu.VMEM((NW * 16,), _f32),
        ],
    )(fpart, epart, ivec)

    return jnp.sum(eout), force[:N_NODES * 3].reshape(N_NODES, 3)
